# baseline (speedup 1.0000x reference)
import functools

import jax
import jax.numpy as jnp
from jax import lax
from jax.experimental import pallas as pl
from jax.experimental.pallas import tpu as pltpu
from jax.experimental.pallas import tpu_sc as plsc

N = 10000
NP = 10240
D = 128
DH = D // 2
E = 320000
C = 32
EP = 327680
NCHUNK = EP // C
CPT = NCHUNK // 16
CPT_DEG = NCHUNK // 32
RPT = NP // 16
NB = 10

_mesh = plsc.VectorSubcoreMesh(core_axis_name="c", subcore_axis_name="s")


@functools.partial(
    pl.kernel,
    out_type=jax.ShapeDtypeStruct((2, NP), jnp.float32),
    mesh=_mesh,
    scratch_types=[
        pltpu.VMEM((CPT_DEG, C), jnp.int32),
        pltpu.VMEM((C,), jnp.float32),
        pltpu.VMEM((RPT,), jnp.float32),
        pltpu.VMEM_SHARED((NP,), jnp.float32),
    ],
)
def _deg_kernel(dst_hbm, out_hbm, idxd_v, ones_v, zeros_v, acc):
    c = lax.axis_index("c")
    s = lax.axis_index("s")
    wid = c * 16 + s

    def _fill(i, _):
        ones_v[pl.ds(i * 16, 16)] = jnp.full((16,), 1.0, jnp.float32)
        return 0
    lax.fori_loop(0, C // 16, _fill, 0)

    def _zero(i, _):
        zeros_v[pl.ds(i * 16, 16)] = jnp.zeros((16,), jnp.float32)
        return 0
    lax.fori_loop(0, RPT // 16, _zero, 0)

    pltpu.sync_copy(zeros_v, acc.at[pl.ds(s * RPT, RPT)])
    plsc.subcore_barrier()

    pltpu.sync_copy(dst_hbm.at[pl.ds(wid * CPT_DEG, CPT_DEG)], idxd_v)

    def _accum(j, _):
        pltpu.sync_copy(ones_v, acc.at[idxd_v.at[j]], add=True)
        return 0
    lax.fori_loop(0, CPT_DEG, _accum, 0)

    plsc.subcore_barrier()
    pltpu.sync_copy(acc.at[pl.ds(s * RPT, RPT)],
                    out_hbm.at[c, pl.ds(s * RPT, RPT)])


@functools.partial(
    pl.kernel,
    out_type=jax.ShapeDtypeStruct((2, NP, DH), jnp.float32),
    mesh=_mesh,
    scratch_types=[
        pltpu.VMEM((CPT, C), jnp.int32),
        pltpu.VMEM((CPT, C), jnp.int32),
        [pltpu.VMEM((C, DH), jnp.float32)] * NB,
        pltpu.VMEM_SHARED((NP, DH), jnp.float32),
        [pltpu.SemaphoreType.DMA] * NB,
        [pltpu.SemaphoreType.DMA] * NB,
    ],
    compiler_params=pltpu.CompilerParams(use_tc_tiling_on_sc=False),
)
def _agg_kernel(src_hbm, dst_hbm, h_hbm, out_hbm,
                idxs_v, idxd_v, rows, acc, gsems, ssems):
    c = lax.axis_index("c")
    s = lax.axis_index("s")
    rz = rows[NB - 1]
    hc = h_hbm.at[c]

    def _gather(j, b):
        pltpu.async_copy(hc.at[idxs_v.at[j]], rows[b], gsems[b])

    def _wait_gather(j, b):
        pltpu.make_async_copy(hc.at[idxs_v.at[j]], rows[b], gsems[b]).wait()

    pltpu.sync_copy(src_hbm.at[pl.ds(s * CPT, CPT)], idxs_v)
    pltpu.sync_copy(dst_hbm.at[pl.ds(s * CPT, CPT)], idxd_v)
    for b in range(NB - 1):
        _gather(b, b)

    def _zero_row(i, _):
        for jj in range(DH // 16):
            rz[i, pl.ds(jj * 16, 16)] = jnp.zeros((16,), jnp.float32)
        return 0
    lax.fori_loop(0, C, _zero_row, 0)

    def _zero_acc(k, _):
        pltpu.sync_copy(rz, acc.at[pl.ds(s * RPT + k * C, C)])
        return 0
    lax.fori_loop(0, RPT // C, _zero_acc, 0)
    plsc.subcore_barrier()
    _gather(NB - 1, NB - 1)


    def _window(i, _):
        j0 = i * NB
        descs = []
        for b in range(NB):
            _wait_gather(j0 + b, b)
            descs.append(pltpu.async_copy(rows[b], acc.at[idxd_v.at[j0 + b]],
                                          ssems[b], add=True))
        for b in range(NB):
            descs[b].wait()
            _gather(j0 + NB + b, b)
        return 0
    lax.fori_loop(0, CPT // NB - 1, _window, 0)

    j0 = CPT - NB
    descs = []
    for b in range(NB):
        _wait_gather(j0 + b, b)
        descs.append(pltpu.async_copy(rows[b], acc.at[idxd_v.at[j0 + b]],
                                      ssems[b], add=True))
    for b in range(NB):
        descs[b].wait()

    plsc.subcore_barrier()
    pltpu.sync_copy(acc.at[pl.ds(s * RPT, RPT)],
                    out_hbm.at[c, pl.ds(s * RPT, RPT)])


_BLK = 1024
_GRID = NP // _BLK


def _dinv_of(deg_ref):
    deg = deg_ref[0] + deg_ref[1]
    return jnp.where(deg > 0, lax.rsqrt(deg), 0.0)


def _split_store(o_ref, res):
    o_ref[0] = res[:, :DH]
    o_ref[1] = res[:, DH:]


def _mm1_body(x_ref, w_ref, deg_ref, o_ref):
    dinv = _dinv_of(deg_ref)
    _split_store(o_ref, jnp.dot(x_ref[...], w_ref[...],
                                preferred_element_type=jnp.float32) * dinv)


def _mm2_body(p_ref, deg_ref, b_ref, w_ref, o_ref):
    dinv = _dinv_of(deg_ref)
    a1 = jnp.concatenate([p_ref[0], p_ref[1]], axis=1) * dinv + b_ref[...]
    r1 = jnp.maximum(a1, 0.0)
    _split_store(o_ref, jnp.dot(r1, w_ref[...],
                                preferred_element_type=jnp.float32) * dinv)


def _fin_body(p_ref, deg_ref, b_ref, o_ref):
    dinv = _dinv_of(deg_ref)
    o_ref[...] = (jnp.concatenate([p_ref[0], p_ref[1]], axis=1) * dinv
                  + b_ref[...])


_row_spec = pl.BlockSpec((_BLK, D), lambda i: (i, 0))
_w_spec = pl.BlockSpec((D, D), lambda i: (0, 0))
_deg_spec = pl.BlockSpec((2, _BLK, 1), lambda i: (0, i, 0))
_half_spec = pl.BlockSpec((2, _BLK, DH), lambda i: (0, i, 0))
_b_spec = pl.BlockSpec((1, D), lambda i: (0, 0))
_half_sds = jax.ShapeDtypeStruct((2, NP, DH), jnp.float32)

_mm1 = pl.pallas_call(
    _mm1_body, grid=(_GRID,), out_shape=_half_sds,
    in_specs=[_row_spec, _w_spec, _deg_spec], out_specs=_half_spec)
_mm2 = pl.pallas_call(
    _mm2_body, grid=(_GRID,), out_shape=_half_sds,
    in_specs=[_half_spec, _deg_spec, _b_spec, _w_spec], out_specs=_half_spec)
_fin = pl.pallas_call(
    _fin_body, grid=(_GRID,),
    out_shape=jax.ShapeDtypeStruct((NP, D), jnp.float32),
    in_specs=[_half_spec, _deg_spec, _b_spec], out_specs=_row_spec)


@jax.jit
def kernel(x, edge_index, W1, b1, W2, b2):
    ei = edge_index.astype(jnp.int32)
    src = jnp.concatenate([ei[0], jnp.zeros((EP - E,), jnp.int32)])
    dst = jnp.concatenate([ei[1], jnp.full((EP - E,), N, jnp.int32)])
    src2d = src.reshape(NCHUNK, C)
    dst2d = dst.reshape(NCHUNK, C)
    xp = jnp.concatenate([x, jnp.zeros((NP - N, D), x.dtype)])

    degp = _deg_kernel(dst2d)
    degp3 = degp.reshape(2, NP, 1)
    h1 = _mm1(xp, W1, degp3)
    p1 = _agg_kernel(src2d, dst2d, h1)
    h2 = _mm2(p1, degp3, b1.reshape(1, D), W2)
    p2 = _agg_kernel(src2d, dst2d, h2)
    out = _fin(p2, degp3, b2.reshape(1, D))
    return out[:N]


# speedup vs baseline: 1.3270x; 1.3270x over previous
import functools

import jax
import jax.numpy as jnp
from jax import lax
from jax.experimental import pallas as pl
from jax.experimental.pallas import tpu as pltpu
from jax.experimental.pallas import tpu_sc as plsc

N = 10000
NP = 10240
D = 128
DH = D // 2
E = 320000
C = 80
EP = 327680
NCHUNK = EP // C
CPT = NCHUNK // 16
CPT_DEG = NCHUNK // 32
RPT = NP // 16
NB = 8

_mesh = plsc.VectorSubcoreMesh(core_axis_name="c", subcore_axis_name="s")


@functools.partial(
    pl.kernel,
    out_type=jax.ShapeDtypeStruct((2, NP), jnp.float32),
    mesh=_mesh,
    scratch_types=[
        pltpu.VMEM((CPT_DEG, C), jnp.int32),
        pltpu.VMEM((C,), jnp.float32),
        pltpu.VMEM((RPT,), jnp.float32),
        pltpu.VMEM_SHARED((NP,), jnp.float32),
    ],
)
def _deg_kernel(dst_hbm, out_hbm, idxd_v, ones_v, zeros_v, acc):
    c = lax.axis_index("c")
    s = lax.axis_index("s")
    wid = c * 16 + s

    def _fill(i, _):
        ones_v[pl.ds(i * 16, 16)] = jnp.full((16,), 1.0, jnp.float32)
        return 0
    lax.fori_loop(0, C // 16, _fill, 0)

    def _zero(i, _):
        zeros_v[pl.ds(i * 16, 16)] = jnp.zeros((16,), jnp.float32)
        return 0
    lax.fori_loop(0, RPT // 16, _zero, 0)

    pltpu.sync_copy(zeros_v, acc.at[pl.ds(s * RPT, RPT)])
    plsc.subcore_barrier()

    pltpu.sync_copy(dst_hbm.at[pl.ds(wid * CPT_DEG, CPT_DEG)], idxd_v)

    def _accum(j, _):
        pltpu.sync_copy(ones_v, acc.at[idxd_v.at[j]], add=True)
        return 0
    lax.fori_loop(0, CPT_DEG, _accum, 0)

    plsc.subcore_barrier()
    pltpu.sync_copy(acc.at[pl.ds(s * RPT, RPT)],
                    out_hbm.at[c, pl.ds(s * RPT, RPT)])


@functools.partial(
    pl.kernel,
    out_type=jax.ShapeDtypeStruct((2, NP, DH), jnp.float32),
    mesh=_mesh,
    scratch_types=[
        pltpu.VMEM((CPT, C), jnp.int32),
        pltpu.VMEM((CPT, C), jnp.int32),
        [pltpu.VMEM((C, DH), jnp.float32)] * NB,
        pltpu.VMEM_SHARED((NP, DH), jnp.float32),
        [pltpu.SemaphoreType.DMA] * NB,
        [pltpu.SemaphoreType.DMA] * NB,
    ],
    compiler_params=pltpu.CompilerParams(use_tc_tiling_on_sc=False),
)
def _agg_kernel(src_hbm, dst_hbm, h_hbm, out_hbm,
                idxs_v, idxd_v, rows, acc, gsems, ssems):
    c = lax.axis_index("c")
    s = lax.axis_index("s")
    rz = rows[NB - 1]
    hc = h_hbm.at[c]

    def _gather(j, b):
        pltpu.async_copy(hc.at[idxs_v.at[j]], rows[b], gsems[b])

    def _wait_gather(j, b):
        pltpu.make_async_copy(hc.at[idxs_v.at[j]], rows[b], gsems[b]).wait()

    pltpu.sync_copy(src_hbm.at[pl.ds(s * CPT, CPT)], idxs_v)
    pltpu.sync_copy(dst_hbm.at[pl.ds(s * CPT, CPT)], idxd_v)
    for b in range(NB - 1):
        _gather(b, b)

    def _zero_row(i, _):
        for jj in range(DH // 16):
            rz[i, pl.ds(jj * 16, 16)] = jnp.zeros((16,), jnp.float32)
        return 0
    lax.fori_loop(0, C, _zero_row, 0)

    def _zero_acc(k, _):
        pltpu.sync_copy(rz, acc.at[pl.ds(s * RPT + k * C, C)])
        return 0
    lax.fori_loop(0, RPT // C, _zero_acc, 0)
    plsc.subcore_barrier()
    _gather(NB - 1, NB - 1)


    def _window(i, _):
        j0 = i * NB
        descs = []
        for b in range(NB):
            _wait_gather(j0 + b, b)
            descs.append(pltpu.async_copy(rows[b], acc.at[idxd_v.at[j0 + b]],
                                          ssems[b], add=True))
        for b in range(NB):
            descs[b].wait()
            _gather(j0 + NB + b, b)
        return 0
    lax.fori_loop(0, CPT // NB - 1, _window, 0)

    j0 = CPT - NB
    descs = []
    for b in range(NB):
        _wait_gather(j0 + b, b)
        descs.append(pltpu.async_copy(rows[b], acc.at[idxd_v.at[j0 + b]],
                                      ssems[b], add=True))
    for b in range(NB):
        descs[b].wait()

    plsc.subcore_barrier()
    pltpu.sync_copy(acc.at[pl.ds(s * RPT, RPT)],
                    out_hbm.at[c, pl.ds(s * RPT, RPT)])


_BLK = 1024
_GRID = NP // _BLK


def _dinv_of(deg_ref):
    deg = deg_ref[0] + deg_ref[1]
    return jnp.where(deg > 0, lax.rsqrt(deg), 0.0)


def _split_store(o_ref, res):
    o_ref[0] = res[:, :DH]
    o_ref[1] = res[:, DH:]


def _mm1_body(x_ref, w_ref, deg_ref, o_ref):
    dinv = _dinv_of(deg_ref)
    _split_store(o_ref, jnp.dot(x_ref[...], w_ref[...],
                                preferred_element_type=jnp.float32) * dinv)


def _mm2_body(p_ref, deg_ref, b_ref, w_ref, o_ref):
    dinv = _dinv_of(deg_ref)
    a1 = jnp.concatenate([p_ref[0], p_ref[1]], axis=1) * dinv + b_ref[...]
    r1 = jnp.maximum(a1, 0.0)
    _split_store(o_ref, jnp.dot(r1, w_ref[...],
                                preferred_element_type=jnp.float32) * dinv)


def _fin_body(p_ref, deg_ref, b_ref, o_ref):
    dinv = _dinv_of(deg_ref)
    o_ref[...] = (jnp.concatenate([p_ref[0], p_ref[1]], axis=1) * dinv
                  + b_ref[...])


_row_spec = pl.BlockSpec((_BLK, D), lambda i: (i, 0))
_w_spec = pl.BlockSpec((D, D), lambda i: (0, 0))
_deg_spec = pl.BlockSpec((2, _BLK, 1), lambda i: (0, i, 0))
_half_spec = pl.BlockSpec((2, _BLK, DH), lambda i: (0, i, 0))
_b_spec = pl.BlockSpec((1, D), lambda i: (0, 0))
_half_sds = jax.ShapeDtypeStruct((2, NP, DH), jnp.float32)

_mm1 = pl.pallas_call(
    _mm1_body, grid=(_GRID,), out_shape=_half_sds,
    in_specs=[_row_spec, _w_spec, _deg_spec], out_specs=_half_spec)
_mm2 = pl.pallas_call(
    _mm2_body, grid=(_GRID,), out_shape=_half_sds,
    in_specs=[_half_spec, _deg_spec, _b_spec, _w_spec], out_specs=_half_spec)
_fin = pl.pallas_call(
    _fin_body, grid=(_GRID,),
    out_shape=jax.ShapeDtypeStruct((NP, D), jnp.float32),
    in_specs=[_half_spec, _deg_spec, _b_spec], out_specs=_row_spec)


@jax.jit
def kernel(x, edge_index, W1, b1, W2, b2):
    ei = edge_index.astype(jnp.int32)
    src = jnp.concatenate([ei[0], jnp.zeros((EP - E,), jnp.int32)])
    dst = jnp.concatenate([ei[1], jnp.full((EP - E,), N, jnp.int32)])
    src2d = src.reshape(NCHUNK, C)
    dst2d = dst.reshape(NCHUNK, C)
    xp = jnp.concatenate([x, jnp.zeros((NP - N, D), x.dtype)])

    degp = _deg_kernel(dst2d)
    degp3 = degp.reshape(2, NP, 1)
    h1 = _mm1(xp, W1, degp3)
    p1 = _agg_kernel(src2d, dst2d, h1)
    h2 = _mm2(p1, degp3, b1.reshape(1, D), W2)
    p2 = _agg_kernel(src2d, dst2d, h2)
    out = _fin(p2, degp3, b2.reshape(1, D))
    return out[:N]
